# baseline (speedup 1.0000x reference)
import functools

import jax
import jax.numpy as jnp
from jax.experimental import pallas as pl
from jax.experimental.pallas import tpu as pltpu

_B, _N, _D_IN, _D_H, _D_OUT = 256, 64, 32, 64, 32
_BT = 32
_H = _N // 2
_EPS = 1e-5


def _fused_kernel(x_ref, eW1a2_ref, eW1b_ref, eb1_ref, eWsum_ref, b1d_ref,
                  Wc_ref, bc_ref, Ag_ref, betad_ref,
                  W3v_ref, W3d_ref, b3d_ref,
                  nW1x_ref, nW1ad_ref, nb1d_ref,
                  Wcn_ref, bcn_ref, Agn_ref, betand_ref,
                  nW3d_ref, nb3d_ref, out_ref):
    x = x_ref[...]
    xf = x.reshape(_BT * _N, _D_IN)
    xlo = x[:, :_H, :].reshape(_BT * _H, _D_IN)
    xhi = x[:, _H:, :].reshape(_BT * _H, _D_IN)
    p2 = xf @ eW1a2_ref[...]
    q2 = jnp.concatenate([xlo @ eW1b_ref[...] + eb1_ref[...],
                          xhi @ eW1b_ref[...] + eb1_ref[...]], axis=-1)
    q2 = q2.reshape(_BT, _H, 2 * _D_H)
    h1 = jnp.maximum(p2.reshape(_BT, _N, 1, 2 * _D_H) + q2[:, None], 0.0)
    h1 = h1.reshape(_BT * _N * _H, 2 * _D_H)
    d = h1 @ Wc_ref[...] + bc_ref[...]
    var = (d * d) @ Ag_ref[...]
    h3 = jnp.maximum(d * jax.lax.rsqrt(var + _EPS) + betad_ref[...], 0.0)
    s = jnp.sum(h3.reshape(_BT, _N, _H, 2 * _D_H), axis=2)
    s = s.reshape(_BT, _N, 2 * _D_H)
    slo = s[:, :_H, :].reshape(_BT * _H, 2 * _D_H)
    shi = s[:, _H:, :].reshape(_BT * _H, 2 * _D_H)
    agg_full = jnp.concatenate([slo @ W3v_ref[...], shi @ W3v_ref[...]],
                               axis=-1) + _N * b3d_ref[...]
    d1 = jnp.maximum(jnp.concatenate([xlo @ eWsum_ref[...],
                                      xhi @ eWsum_ref[...]], axis=-1)
                     + b1d_ref[...], 0.0)
    dc = d1 @ Wc_ref[...] + bc_ref[...]
    dvar = (dc * dc) @ Ag_ref[...]
    dh3 = jnp.maximum(dc * jax.lax.rsqrt(dvar + _EPS) + betad_ref[...], 0.0)
    d3 = dh3 @ W3d_ref[...] + b3d_ref[...]
    agg = agg_full - d3
    g1 = jnp.maximum(jnp.concatenate([xlo @ nW1x_ref[...],
                                      xhi @ nW1x_ref[...]], axis=-1)
                     + agg @ nW1ad_ref[...] + nb1d_ref[...], 0.0)
    gc = g1 @ Wcn_ref[...] + bcn_ref[...]
    gvar = (gc * gc) @ Agn_ref[...]
    gh = jnp.maximum(gc * jax.lax.rsqrt(gvar + _EPS) + betand_ref[...], 0.0)
    out2 = gh @ nW3d_ref[...] + nb3d_ref[...]
    out2 = out2.reshape(_BT, _H, 2 * _D_OUT)
    out_ref[:, :_H, :] = out2[:, :, :_D_OUT]
    out_ref[:, _H:, :] = out2[:, :, _D_OUT:]


def _ln_fold(W, b, g):
    f32 = jnp.float32
    z = jnp.zeros((_D_H, _D_H), f32)
    Wd = jnp.block([[W, z], [z, W]])
    bd = jnp.concatenate([b, b]).reshape(1, -1)
    ones = jnp.ones((_D_H, _D_H), f32) / _D_H
    A = jnp.block([[ones, z], [z, ones]])
    gd = jnp.concatenate([g, g])
    Wc = (Wd - Wd @ A) * gd[None, :]
    bc = (bd - bd @ A) * gd[None, :]
    Ag = A / (gd * gd)[:, None]
    return Wc, bc, Ag


@functools.partial(jax.jit, static_argnames=("interpret",))
def _run(states, eW1, eb1, eW2, eb2, eg, ebeta, eW3, eb3,
         nW1, nb1, nW2, nb2, ng, nbeta, nW3, nb3, interpret=False):
    f32 = jnp.float32
    row = lambda v: v.reshape(1, -1)
    dbl = lambda v: row(jnp.concatenate([v, v]))
    eW1a, eW1b = eW1[:_D_IN], eW1[_D_IN:]
    eW1a2 = jnp.concatenate([eW1a, eW1a], axis=1)
    eWsum = eW1a + eW1b
    Wc, bc, Ag = _ln_fold(eW2, eb2, eg)
    W3v = jnp.concatenate([eW3, eW3], axis=0)
    z64 = jnp.zeros((_D_H, _D_H), f32)
    W3d = jnp.block([[eW3, z64], [z64, eW3]])
    nW1x, nW1a = nW1[:_D_IN], nW1[_D_IN:]
    nW1ad = jnp.block([[nW1a, z64], [z64, nW1a]])
    Wcn, bcn, Agn = _ln_fold(nW2, nb2, ng)
    z63 = jnp.zeros((_D_H, _D_OUT), f32)
    nW3d = jnp.block([[nW3, z63], [z63, nW3]])
    weights = (eW1a2, eW1b, row(eb1), eWsum, dbl(eb1),
               Wc, bc, Ag, dbl(ebeta),
               W3v, W3d, dbl(eb3),
               nW1x, nW1ad, dbl(nb1),
               Wcn, bcn, Agn, dbl(nbeta),
               nW3d, dbl(nb3))
    full = lambda w: pl.BlockSpec(w.shape, lambda b: (0,) * w.ndim)
    out = pl.pallas_call(
        _fused_kernel,
        grid=(_B // _BT,),
        in_specs=[pl.BlockSpec((_BT, _N, _D_IN), lambda b: (b, 0, 0))]
                 + [full(w) for w in weights],
        out_specs=pl.BlockSpec((_BT, _N, _D_OUT), lambda b: (b, 0, 0)),
        out_shape=jax.ShapeDtypeStruct((_B, _N, _D_OUT), f32),
        compiler_params=pltpu.CompilerParams(
            dimension_semantics=("parallel",)),
        interpret=interpret,
    )(states, *weights)
    return out


def kernel(states, action, viz, eW1, eb1, eW2, eb2, eg, ebeta, eW3, eb3,
           nW1, nb1, nW2, nb2, ng, nbeta, nW3, nb3):
    out = _run(states, eW1, eb1, eW2, eb2, eg, ebeta, eW3, eb3,
               nW1, nb1, nW2, nb2, ng, nbeta, nW3, nb3)
    return (out, action, viz)


# speedup vs baseline: 1.1084x; 1.1084x over previous
import functools

import jax
import jax.numpy as jnp
from jax.experimental import pallas as pl
from jax.experimental.pallas import tpu as pltpu

_B, _N, _D_IN, _D_H, _D_OUT = 256, 64, 32, 64, 32
_BT = 32
_H = _N // 2
_EPS = 1e-5


def _bdiag(w):
    z = jnp.zeros_like(w)
    return jnp.concatenate([jnp.concatenate([w, z], axis=1),
                            jnp.concatenate([z, w], axis=1)], axis=0)


def _dbl(rowv):
    return jnp.concatenate([rowv, rowv], axis=1)


def _ln_fold(W, brow, grow, gcol):
    Wc64 = (W - jnp.mean(W, axis=1, keepdims=True)) * grow
    bc64 = (brow - jnp.mean(brow, axis=1, keepdims=True)) * grow
    inv = (1.0 / (_D_H * gcol * gcol)) * jnp.ones((1, _D_H), jnp.float32)
    return _bdiag(Wc64), _dbl(bc64), _bdiag(inv)


def _fused_kernel(x_ref, eW1_ref, eb1_ref, eW2_ref, eb2_ref, eg_ref,
                  egc_ref, ebeta_ref, eW3_ref, eb3_ref, nW1_ref, nb1_ref,
                  nW2_ref, nb2_ref, ng_ref, ngc_ref, nbeta_ref, nW3_ref,
                  nb3_ref, out_ref):
    w1 = eW1_ref[...]
    eW1a, eW1b = w1[:_D_IN], w1[_D_IN:]
    eW1a2 = jnp.concatenate([eW1a, eW1a], axis=1)
    eWsum = eW1a + eW1b
    eb1 = eb1_ref[...]
    b1d = _dbl(eb1)
    Wc, bc, Ag = _ln_fold(eW2_ref[...], eb2_ref[...], eg_ref[...],
                          egc_ref[...])
    betad = _dbl(ebeta_ref[...])
    w3 = eW3_ref[...]
    W3v = jnp.concatenate([w3, w3], axis=0)
    W3d = _bdiag(w3)
    b3d = _dbl(eb3_ref[...])
    nw1 = nW1_ref[...]
    nW1x, nW1a = nw1[:_D_IN], nw1[_D_IN:]
    nW1ad = _bdiag(nW1a)
    nb1d = _dbl(nb1_ref[...])
    Wcn, bcn, Agn = _ln_fold(nW2_ref[...], nb2_ref[...], ng_ref[...],
                             ngc_ref[...])
    betand = _dbl(nbeta_ref[...])
    nW3d = _bdiag(nW3_ref[...])
    nb3d = _dbl(nb3_ref[...])
    x = x_ref[...]
    xf = x.reshape(_BT * _N, _D_IN)
    xlo = x[:, :_H, :].reshape(_BT * _H, _D_IN)
    xhi = x[:, _H:, :].reshape(_BT * _H, _D_IN)
    p2 = xf @ eW1a2
    q2 = jnp.concatenate([xlo @ eW1b + eb1, xhi @ eW1b + eb1], axis=-1)
    q2 = q2.reshape(_BT, _H, 2 * _D_H)
    h1 = jnp.maximum(p2.reshape(_BT, _N, 1, 2 * _D_H) + q2[:, None], 0.0)
    h1 = h1.reshape(_BT * _N * _H, 2 * _D_H)
    d = h1 @ Wc + bc
    var = (d * d) @ Ag
    h3 = jnp.maximum(d * jax.lax.rsqrt(var + _EPS) + betad, 0.0)
    s = jnp.sum(h3.reshape(_BT, _N, _H, 2 * _D_H), axis=2)
    s = s.reshape(_BT, _N, 2 * _D_H)
    slo = s[:, :_H, :].reshape(_BT * _H, 2 * _D_H)
    shi = s[:, _H:, :].reshape(_BT * _H, 2 * _D_H)
    agg_full = jnp.concatenate([slo @ W3v, shi @ W3v], axis=-1) \
        + _N * b3d
    d1 = jnp.maximum(jnp.concatenate([xlo @ eWsum, xhi @ eWsum], axis=-1)
                     + b1d, 0.0)
    dc = d1 @ Wc + bc
    dvar = (dc * dc) @ Ag
    dh3 = jnp.maximum(dc * jax.lax.rsqrt(dvar + _EPS) + betad, 0.0)
    d3 = dh3 @ W3d + b3d
    agg = agg_full - d3
    g1 = jnp.maximum(jnp.concatenate([xlo @ nW1x, xhi @ nW1x], axis=-1)
                     + agg @ nW1ad + nb1d, 0.0)
    gc = g1 @ Wcn + bcn
    gvar = (gc * gc) @ Agn
    gh = jnp.maximum(gc * jax.lax.rsqrt(gvar + _EPS) + betand, 0.0)
    out2 = gh @ nW3d + nb3d
    out2 = out2.reshape(_BT, _H, 2 * _D_OUT)
    out_ref[:, :_H, :] = out2[:, :, :_D_OUT]
    out_ref[:, _H:, :] = out2[:, :, _D_OUT:]


@functools.partial(jax.jit, static_argnames=("interpret",))
def _run(states, eW1, eb1, eW2, eb2, eg, ebeta, eW3, eb3,
         nW1, nb1, nW2, nb2, ng, nbeta, nW3, nb3, interpret=False):
    row = lambda v: v.reshape(1, -1)
    weights = (eW1, row(eb1), eW2, row(eb2), row(eg), eg.reshape(-1, 1),
               row(ebeta), eW3, row(eb3), nW1, row(nb1), nW2, row(nb2),
               row(ng), ng.reshape(-1, 1), row(nbeta), nW3, row(nb3))
    full = lambda w: pl.BlockSpec(w.shape, lambda b: (0,) * w.ndim)
    out = pl.pallas_call(
        _fused_kernel,
        grid=(_B // _BT,),
        in_specs=[pl.BlockSpec((_BT, _N, _D_IN), lambda b: (b, 0, 0))]
                 + [full(w) for w in weights],
        out_specs=pl.BlockSpec((_BT, _N, _D_OUT), lambda b: (b, 0, 0)),
        out_shape=jax.ShapeDtypeStruct((_B, _N, _D_OUT), jnp.float32),
        compiler_params=pltpu.CompilerParams(
            dimension_semantics=("parallel",)),
        interpret=interpret,
    )(states, *weights)
    return out


def kernel(states, action, viz, eW1, eb1, eW2, eb2, eg, ebeta, eW3, eb3,
           nW1, nb1, nW2, nb2, ng, nbeta, nW3, nb3):
    out = _run(states, eW1, eb1, eW2, eb2, eg, ebeta, eW3, eb3,
               nW1, nb1, nW2, nb2, ng, nbeta, nW3, nb3)
    return (out, action, viz)


# speedup vs baseline: 1.1245x; 1.0146x over previous
import functools

import jax
import jax.numpy as jnp
from jax.experimental import pallas as pl
from jax.experimental.pallas import tpu as pltpu

_B, _N, _D_IN, _D_H, _D_OUT = 256, 64, 32, 64, 32
_BT = 32
_H = _N // 2
_EPS = 1e-5


def _bdiag(w):
    z = jnp.zeros_like(w)
    return jnp.concatenate([jnp.concatenate([w, z], axis=1),
                            jnp.concatenate([z, w], axis=1)], axis=0)


def _dbl(rowv):
    return jnp.concatenate([rowv, rowv], axis=1)


def _ln_fold(W, brow, grow, gcol):
    Wc64 = (W - jnp.mean(W, axis=1, keepdims=True)) * grow
    bc64 = (brow - jnp.mean(brow, axis=1, keepdims=True)) * grow
    inv = (1.0 / (_D_H * gcol * gcol)) * jnp.ones((1, _D_H), jnp.float32)
    return _bdiag(Wc64), _dbl(bc64), _bdiag(inv)


def _fused_kernel(x_ref, eW1_ref, eb1_ref, eW2_ref, eb2_ref, eg_ref,
                  egc_ref, ebeta_ref, eW3_ref, eb3_ref, nW1_ref, nb1_ref,
                  nW2_ref, nb2_ref, ng_ref, ngc_ref, nbeta_ref, nW3_ref,
                  nb3_ref, out_ref):
    w1 = eW1_ref[...]
    eW1a, eW1b = w1[:_D_IN], w1[_D_IN:]
    eW1a2 = jnp.concatenate([eW1a, eW1a], axis=1)
    eWsum = eW1a + eW1b
    eb1 = eb1_ref[...]
    b1d = _dbl(eb1)
    Wc, bc, Ag = _ln_fold(eW2_ref[...], eb2_ref[...], eg_ref[...],
                          egc_ref[...])
    betad = _dbl(ebeta_ref[...])
    w3 = eW3_ref[...]
    W3v = jnp.concatenate([w3, w3], axis=0)
    W3d = _bdiag(w3)
    b3d = _dbl(eb3_ref[...])
    nw1 = nW1_ref[...]
    nW1x, nW1a = nw1[:_D_IN], nw1[_D_IN:]
    nW1ad = _bdiag(nW1a)
    nb1d = _dbl(nb1_ref[...])
    Wcn, bcn, Agn = _ln_fold(nW2_ref[...], nb2_ref[...], ng_ref[...],
                             ngc_ref[...])
    betand = _dbl(nbeta_ref[...])
    nW3d = _bdiag(nW3_ref[...])
    nb3d = _dbl(nb3_ref[...])
    x = x_ref[...]
    xf = x.reshape(_BT * _N, _D_IN)
    xlo = x[:, :_H, :].reshape(_BT * _H, _D_IN)
    xhi = x[:, _H:, :].reshape(_BT * _H, _D_IN)
    p2 = (xf @ eW1a2).astype(jnp.bfloat16)
    q2 = jnp.concatenate([xlo @ eW1b + eb1, xhi @ eW1b + eb1], axis=-1)
    q2 = q2.astype(jnp.bfloat16).reshape(_BT, _H, 2 * _D_H)
    h1 = jnp.maximum(p2.reshape(_BT, _N, 1, 2 * _D_H) + q2[:, None],
                     jnp.bfloat16(0.0))
    h1 = h1.reshape(_BT * _N * _H, 2 * _D_H)
    d = jax.lax.dot_general(h1, Wc.astype(jnp.bfloat16),
                            (((1,), (0,)), ((), ())),
                            preferred_element_type=jnp.float32) + bc
    var = (d * d) @ Ag
    h3 = jnp.maximum(d * jax.lax.rsqrt(var + _EPS) + betad, 0.0)
    s = jnp.sum(h3.reshape(_BT, _N, _H, 2 * _D_H), axis=2)
    s = s.reshape(_BT, _N, 2 * _D_H)
    slo = s[:, :_H, :].reshape(_BT * _H, 2 * _D_H)
    shi = s[:, _H:, :].reshape(_BT * _H, 2 * _D_H)
    agg_full = jnp.concatenate([slo @ W3v, shi @ W3v], axis=-1) \
        + _N * b3d
    d1 = jnp.maximum(jnp.concatenate([xlo @ eWsum, xhi @ eWsum], axis=-1)
                     + b1d, 0.0)
    dc = d1 @ Wc + bc
    dvar = (dc * dc) @ Ag
    dh3 = jnp.maximum(dc * jax.lax.rsqrt(dvar + _EPS) + betad, 0.0)
    d3 = dh3 @ W3d + b3d
    agg = agg_full - d3
    g1 = jnp.maximum(jnp.concatenate([xlo @ nW1x, xhi @ nW1x], axis=-1)
                     + agg @ nW1ad + nb1d, 0.0)
    gc = g1 @ Wcn + bcn
    gvar = (gc * gc) @ Agn
    gh = jnp.maximum(gc * jax.lax.rsqrt(gvar + _EPS) + betand, 0.0)
    out2 = gh @ nW3d + nb3d
    out2 = out2.reshape(_BT, _H, 2 * _D_OUT)
    out_ref[:, :_H, :] = out2[:, :, :_D_OUT]
    out_ref[:, _H:, :] = out2[:, :, _D_OUT:]


@functools.partial(jax.jit, static_argnames=("interpret",))
def _run(states, eW1, eb1, eW2, eb2, eg, ebeta, eW3, eb3,
         nW1, nb1, nW2, nb2, ng, nbeta, nW3, nb3, interpret=False):
    row = lambda v: v.reshape(1, -1)
    weights = (eW1, row(eb1), eW2, row(eb2), row(eg), eg.reshape(-1, 1),
               row(ebeta), eW3, row(eb3), nW1, row(nb1), nW2, row(nb2),
               row(ng), ng.reshape(-1, 1), row(nbeta), nW3, row(nb3))
    full = lambda w: pl.BlockSpec(w.shape, lambda b: (0,) * w.ndim)
    out = pl.pallas_call(
        _fused_kernel,
        grid=(_B // _BT,),
        in_specs=[pl.BlockSpec((_BT, _N, _D_IN), lambda b: (b, 0, 0))]
                 + [full(w) for w in weights],
        out_specs=pl.BlockSpec((_BT, _N, _D_OUT), lambda b: (b, 0, 0)),
        out_shape=jax.ShapeDtypeStruct((_B, _N, _D_OUT), jnp.float32),
        compiler_params=pltpu.CompilerParams(
            dimension_semantics=("parallel",)),
        interpret=interpret,
    )(states, *weights)
    return out


def kernel(states, action, viz, eW1, eb1, eW2, eb2, eg, ebeta, eW3, eb3,
           nW1, nb1, nW2, nb2, ng, nbeta, nW3, nb3):
    out = _run(states, eW1, eb1, eW2, eb2, eg, ebeta, eW3, eb3,
               nW1, nb1, nW2, nb2, ng, nbeta, nW3, nb3)
    return (out, action, viz)
